# baseline (speedup 1.0000x reference)
import functools

import jax
import jax.numpy as jnp
from jax import lax
from jax.experimental import pallas as pl
from jax.experimental.pallas import tpu as pltpu
from jax.experimental.pallas import tpu_sc as plsc

VOCAB = 1000000
EMBED = 64
SEQ = 200
BATCH = 1024
N_ROWS = BATCH * SEQ
HALF = EMBED // 2
SCALE = 8.0

NC = 2
NS = 16
NW = NC * NS
SEQ_PER_W = BATCH // NW
IDX_MINOR = 100


def _pe_body(o_ref):
    pos = lax.broadcasted_iota(jnp.int32, (SEQ, HALF), 0).astype(jnp.float32)
    i = lax.broadcasted_iota(jnp.int32, (SEQ, HALF), 1).astype(jnp.float32)
    inv_freq = jnp.exp(i * (-9.210340371976184 / HALF))
    angles = pos * inv_freq
    o_ref[:, :HALF] = jnp.sin(angles)
    o_ref[:, HALF:] = jnp.cos(angles)


def _pe_table():
    return pl.pallas_call(
        _pe_body,
        out_shape=jax.ShapeDtypeStruct((SEQ, EMBED), jnp.float32),
    )()


_mesh = plsc.VectorSubcoreMesh(core_axis_name="c", subcore_axis_name="s")


@functools.partial(
    pl.kernel,
    mesh=_mesh,
    compiler_params=pltpu.CompilerParams(use_tc_tiling_on_sc=False),
    out_type=jax.ShapeDtypeStruct((N_ROWS, EMBED), jnp.float32),
    scratch_types=[
        pltpu.VMEM((2, IDX_MINOR), jnp.int32),
        pltpu.VMEM((SEQ, EMBED), jnp.float32),
        pltpu.VMEM((SEQ, EMBED), jnp.float32),
        pltpu.SemaphoreType.DMA,
    ],
)
def _sc_gather(table_hbm, idx_hbm, pe_hbm, out_hbm, idx_v, rows_v, pe_v, sem):
    wid = lax.axis_index("s") * NC + lax.axis_index("c")
    seq_base = wid * SEQ_PER_W

    pltpu.sync_copy(pe_hbm, pe_v)

    def seq_body(s, _):
        seq = seq_base + s
        row0 = seq * SEQ
        pltpu.sync_copy(idx_hbm.at[pl.ds(2 * seq, 2)], idx_v)
        cp0 = pltpu.async_copy(
            table_hbm.at[idx_v.at[0]], rows_v.at[pl.ds(0, IDX_MINOR)], sem)
        cp1 = pltpu.async_copy(
            table_hbm.at[idx_v.at[1]], rows_v.at[pl.ds(IDX_MINOR, IDX_MINOR)], sem)
        cp0.wait()
        cp1.wait()

        def row_body(r, _):
            for j in range(EMBED // 16):
                sl = pl.ds(j * 16, 16)
                rows_v[r, sl] = rows_v[r, sl] * SCALE + pe_v[r, sl]
            return 0

        lax.fori_loop(0, SEQ, row_body, 0, unroll=2)
        pltpu.sync_copy(rows_v, out_hbm.at[pl.ds(row0, SEQ)])
        return 0

    lax.fori_loop(0, SEQ_PER_W, seq_body, 0)


def kernel(x, table):
    idx = x.astype(jnp.int32).reshape(N_ROWS // IDX_MINOR, IDX_MINOR)
    pe = _pe_table()
    out = _sc_gather(table, idx, pe)
    return out.reshape(BATCH, SEQ, EMBED)


# speedup vs baseline: 1.0423x; 1.0423x over previous
import functools

import jax
import jax.numpy as jnp
from jax import lax
from jax.experimental import pallas as pl
from jax.experimental.pallas import tpu as pltpu
from jax.experimental.pallas import tpu_sc as plsc

VOCAB = 1000000
EMBED = 64
SEQ = 200
BATCH = 1024
N_ROWS = BATCH * SEQ
HALF = EMBED // 2
SCALE = 8.0

NC = 2
NS = 16
NW = NC * NS
SEQ_PER_W = BATCH // NW
IDX_MINOR = 100


def _pe_body(o_ref):
    pos = lax.broadcasted_iota(jnp.int32, (SEQ, HALF), 0).astype(jnp.float32)
    i = lax.broadcasted_iota(jnp.int32, (SEQ, HALF), 1).astype(jnp.float32)
    inv_freq = jnp.exp(i * (-9.210340371976184 / HALF))
    angles = pos * inv_freq
    o_ref[:, :HALF] = jnp.sin(angles)
    o_ref[:, HALF:] = jnp.cos(angles)


def _pe_table():
    return pl.pallas_call(
        _pe_body,
        out_shape=jax.ShapeDtypeStruct((SEQ, EMBED), jnp.float32),
    )()


_mesh = plsc.VectorSubcoreMesh(core_axis_name="c", subcore_axis_name="s")


@functools.partial(
    pl.kernel,
    mesh=_mesh,
    compiler_params=pltpu.CompilerParams(use_tc_tiling_on_sc=False),
    out_type=jax.ShapeDtypeStruct((N_ROWS, EMBED), jnp.float32),
    scratch_types=[
        pltpu.VMEM((2 * SEQ_PER_W, IDX_MINOR), jnp.int32),
        pltpu.VMEM((SEQ, EMBED), jnp.float32),
        pltpu.VMEM((SEQ, EMBED), jnp.float32),
        pltpu.VMEM((SEQ, EMBED), jnp.float32),
        pltpu.VMEM((SEQ, EMBED), jnp.float32),
        pltpu.VMEM((SEQ, EMBED), jnp.float32),
        pltpu.SemaphoreType.DMA,
        pltpu.SemaphoreType.DMA,
        pltpu.SemaphoreType.DMA,
        pltpu.SemaphoreType.DMA,
    ],
)
def _sc_gather(table_hbm, idx_hbm, pe_hbm, out_hbm,
               idx_all, rows0, rows1, ob0, ob1, pe_v, g0, g1, o0, o1):
    rows = (rows0, rows1)
    obuf = (ob0, ob1)
    gsem = (g0, g1)
    osem = (o0, o1)
    wid = lax.axis_index("s") * NC + lax.axis_index("c")
    seq_base = wid * SEQ_PER_W

    pltpu.sync_copy(pe_hbm, pe_v)
    pltpu.sync_copy(idx_hbm.at[pl.ds(2 * seq_base, 2 * SEQ_PER_W)], idx_all)

    def gather_descr(sl, b, j):
        return pltpu.make_async_copy(
            table_hbm.at[idx_all.at[2 * sl + j]],
            rows[b].at[pl.ds(j * IDX_MINOR, IDX_MINOR)],
            gsem[b])

    def out_descr(sl, b):
        return pltpu.make_async_copy(
            obuf[b], out_hbm.at[pl.ds((seq_base + sl) * SEQ, SEQ)], osem[b])

    def fire_gather(sl, b):
        for j in range(2):
            pltpu.async_copy(
                table_hbm.at[idx_all.at[2 * sl + j]],
                rows[b].at[pl.ds(j * IDX_MINOR, IDX_MINOR)],
                gsem[b])

    def compute(b):
        def row_body(r, _):
            for j in range(EMBED // 16):
                sl_ = pl.ds(j * 16, 16)
                obuf[b][r, sl_] = rows[b][r, sl_] * SCALE + pe_v[r, sl_]
            return 0

        lax.fori_loop(0, SEQ, row_body, 0, unroll=4)

    def body(sl, b, first, last):
        for j in range(2):
            gather_descr(sl, b, j).wait()
        if not first:
            out_descr(sl - 2, b).wait()
        compute(b)
        out_descr(sl, b).start()
        if not last:
            fire_gather(sl + 2, b)

    fire_gather(0, 0)
    fire_gather(1, 1)
    body(0, 0, True, False)
    body(1, 1, True, False)

    def t_body(t, _):
        body(2 * t, 0, False, False)
        body(2 * t + 1, 1, False, False)
        return 0

    lax.fori_loop(1, SEQ_PER_W // 2 - 1, t_body, 0)
    body(SEQ_PER_W - 2, 0, False, True)
    body(SEQ_PER_W - 1, 1, False, True)
    out_descr(SEQ_PER_W - 2, 0).wait()
    out_descr(SEQ_PER_W - 1, 1).wait()


def kernel(x, table):
    idx = x.astype(jnp.int32).reshape(N_ROWS // IDX_MINOR, IDX_MINOR)
    pe = _pe_table()
    out = _sc_gather(table, idx, pe)
    return out.reshape(BATCH, SEQ, EMBED)


# speedup vs baseline: 2.4665x; 2.3664x over previous
import functools

import jax
import jax.numpy as jnp
from jax import lax
from jax.experimental import pallas as pl
from jax.experimental.pallas import tpu as pltpu
from jax.experimental.pallas import tpu_sc as plsc

VOCAB = 1000000
EMBED = 64
SEQ = 200
BATCH = 1024
SCALE = 8.0

NC = 2
NS = 16
DIMS_PER_CORE = EMBED // NC
SB = 25
NQ = 4
L = 128


def _pe_body(o_ref):
    r = lax.broadcasted_iota(jnp.int32, (2 * SEQ, DIMS_PER_CORE), 0)
    dl = lax.broadcasted_iota(jnp.int32, (2 * SEQ, DIMS_PER_CORE), 1)
    s = (r % SEQ).astype(jnp.float32)
    inv_freq = jnp.exp(dl.astype(jnp.float32)
                       * (-9.210340371976184 / DIMS_PER_CORE))
    ang = s * inv_freq
    o_ref[...] = jnp.where(r >= SEQ, jnp.cos(ang), jnp.sin(ang))


def _pe_table():
    return pl.pallas_call(
        _pe_body,
        out_shape=jax.ShapeDtypeStruct((2 * SEQ, DIMS_PER_CORE), jnp.float32),
    )()


_mesh = plsc.VectorSubcoreMesh(core_axis_name="c", subcore_axis_name="s")

_PE_TILE = SB * DIMS_PER_CORE


@functools.partial(
    pl.kernel,
    mesh=_mesh,
    out_type=jax.ShapeDtypeStruct((SEQ, EMBED, 8, L), jnp.float32),
    scratch_types=[
        pltpu.VMEM((SB, NQ, L), jnp.int32),
        pltpu.VMEM((SB, NQ, L), jnp.float32),
        pltpu.VMEM((SB, NQ, L), jnp.float32),
        pltpu.VMEM_SHARED((VOCAB,), jnp.float32),
        pltpu.VMEM_SHARED((2 * SEQ * DIMS_PER_CORE,), jnp.float32),
        pltpu.SMEM((_PE_TILE,), jnp.float32),
        pltpu.SemaphoreType.DMA,
        pltpu.SemaphoreType.DMA,
        pltpu.SemaphoreType.DMA,
        pltpu.SemaphoreType.DMA,
    ],
)
def _sc_embed(tabt_hbm, xt_hbm, pe_hbm, out_hbm,
              idx_t, stg0, stg1, row_v, pe_sh, pe_s, rsem, gsem, os0, os1):
    stg = (stg0, stg1)
    osem = (os0, os1)

    cid = lax.axis_index("c")
    sid = lax.axis_index("s")
    i_blk = sid // 2
    j_blk = sid % 2
    s0 = i_blk * SB
    d0 = cid * DIMS_PER_CORE

    def row_descr(d_loc):
        return pltpu.make_async_copy(tabt_hbm.at[d0 + d_loc], row_v, rsem)

    @pl.when(sid == 0)
    def _():
        row_descr(0).start()
        pltpu.sync_copy(pe_hbm, pe_sh)
    pltpu.sync_copy(xt_hbm.at[pl.ds(s0, SB), pl.ds(NQ * j_blk, NQ)], idx_t)
    plsc.subcore_barrier()
    pltpu.sync_copy(
        pe_sh.at[pl.ds((cid * 8 + i_blk) * _PE_TILE, _PE_TILE)], pe_s)

    def gather_descr(sb, sl, q):
        return pltpu.make_async_copy(
            row_v.at[idx_t.at[sl, q]], stg[sb].at[sl, q], gsem)

    def out_descr(d_loc, sb):
        return pltpu.make_async_copy(
            stg[sb],
            out_hbm.at[pl.ds(s0, SB), d0 + d_loc, pl.ds(NQ * j_blk, NQ)],
            osem[sb])

    def run_gathers(sb):
        def sbody(sl, _):
            for q in range(NQ):
                gather_descr(sb, sl, q).start()
            return 0
        lax.fori_loop(0, SB, sbody, 0)

        def wbody(sl, _):
            for q in range(NQ):
                gather_descr(sb, sl, q).wait()
            return 0
        lax.fori_loop(0, SB, wbody, 0)

    def compute(d_loc, sb):
        def sbody(sl, _):
            pv = jnp.full((16,), pe_s[sl * DIMS_PER_CORE + d_loc], jnp.float32)
            for q in range(NQ):
                for v in range(L // 16):
                    w = pl.ds(16 * v, 16)
                    stg[sb][sl, q, w] = stg[sb][sl, q, w] * SCALE + pv
            return 0

        lax.fori_loop(0, SB, sbody, 0)

    def body(d_loc, sb, first, last):
        @pl.when(sid == 0)
        def _():
            row_descr(d_loc).wait()
        plsc.subcore_barrier()
        if not first:
            out_descr(d_loc - 2, sb).wait()
        run_gathers(sb)
        plsc.subcore_barrier()
        if not last:
            @pl.when(sid == 0)
            def _():
                row_descr(d_loc + 1).start()
        compute(d_loc, sb)
        out_descr(d_loc, sb).start()

    body(0, 0, True, False)
    body(1, 1, True, False)

    def t_body(t, _):
        body(2 * t, 0, False, False)
        body(2 * t + 1, 1, False, False)
        return 0

    lax.fori_loop(1, DIMS_PER_CORE // 2 - 1, t_body, 0)
    body(DIMS_PER_CORE - 2, 0, False, False)
    body(DIMS_PER_CORE - 1, 1, False, True)
    out_descr(DIMS_PER_CORE - 2, 0).wait()
    out_descr(DIMS_PER_CORE - 1, 1).wait()


def kernel(x, table):
    tab_t = table.T
    x_t = x.T.astype(jnp.int32).reshape(SEQ, 8, L)
    pe = _pe_table().reshape(-1)
    out4 = _sc_embed(tab_t, x_t, pe)
    return jnp.transpose(out4, (2, 3, 0, 1)).reshape(BATCH, SEQ, EMBED)
